# baseline (speedup 1.0000x reference)
import functools

import jax
import jax.numpy as jnp
from jax import lax
from jax.experimental import pallas as pl
from jax.experimental.pallas import tpu as pltpu
from jax.experimental.pallas import tpu_sc as plsc

_BS, _L, _D = 128, 12, 8192
_NW = 32
_BPW = _BS // _NW
_H = 2048
_UPB = _D // _H
_NU = _BPW * _UPB
_RB = 3


def _sc_body(in_hbm, idx_hbm, drop_hbm, out_hbm, idx_v, drop_v, buf, in_sems, out_sems):
    cid = lax.axis_index("c")
    sid = lax.axis_index("s")
    w = sid * 2 + cid

    pltpu.sync_copy(idx_hbm, idx_v)
    pltpu.sync_copy(drop_hbm, drop_v)

    lane = lax.broadcasted_iota(jnp.int32, (16,), 0)

    def _scalars(b):
        sel = lane == (b % 16)
        idx16 = idx_v[pl.ds((b // 16) * 16, 16)]
        drop16 = drop_v[pl.ds((b // 16) * 16, 16)]
        idx_s = jnp.max(jnp.where(sel, idx16, 0))
        dropped = jnp.max(jnp.where(sel, drop16, 0.0)) > (1.0 - 0.2)
        return idx_s, dropped

    def unit(u):
        j, h = u // _UPB, u % _UPB
        return w * _BPW + j, h

    def _in(u):
        b, h = unit(u)
        return pltpu.make_async_copy(
            in_hbm.at[b, :, pl.ds(h * _H, _H)], buf.at[u % _RB], in_sems.at[u % _RB]
        )

    def _out(u):
        b, h = unit(u)
        return pltpu.make_async_copy(
            buf.at[u % _RB], out_hbm.at[b, :, pl.ds(h * _H, _H)], out_sems.at[u % _RB]
        )

    def _fix(u, idx_s, dropped):
        slot = u % _RB

        @pl.when(dropped)
        def _():
            z = jnp.zeros((16,), jnp.float32)

            def body(i, _):
                buf[slot, idx_s, pl.ds(i * 16, 16)] = z
                return 0

            lax.fori_loop(0, _H // 16, body, 0)

    scal = [_scalars(w * _BPW + j) for j in range(_BPW)]

    _in(0).start()
    for u in range(_NU):
        if u + 1 < _NU:
            if u + 1 >= _RB:
                _out(u + 1 - _RB).wait()
            _in(u + 1).start()
        _in(u).wait()
        idx_s, dropped = scal[u // _UPB]
        _fix(u, idx_s, dropped)
        _out(u).start()
    for u in range(_NU - _RB, _NU):
        _out(u).wait()


@jax.jit
def kernel(inputs, index, drop_rand):
    mesh = plsc.VectorSubcoreMesh(core_axis_name="c", subcore_axis_name="s")
    k = functools.partial(
        pl.kernel,
        mesh=mesh,
        compiler_params=pltpu.CompilerParams(needs_layout_passes=False),
        out_type=jax.ShapeDtypeStruct((_BS, _L, _D), jnp.float32),
        scratch_types=[
            pltpu.VMEM((_BS,), jnp.int32),
            pltpu.VMEM((_BS,), jnp.float32),
            pltpu.VMEM((_RB, _L, _H), jnp.float32),
            pltpu.SemaphoreType.DMA((_RB,)),
            pltpu.SemaphoreType.DMA((_RB,)),
        ],
    )(_sc_body)
    return k(inputs, index, drop_rand)


# speedup vs baseline: 1.6999x; 1.6999x over previous
import jax
import jax.numpy as jnp
from jax.experimental import pallas as pl
from jax.experimental.pallas import tpu as pltpu

_BS, _L, _D = 128, 12, 8192


def _body(idx_ref, drop_ref, in_hbm, out_hbm, zrow, row_sem):
    zrow[...] = jnp.zeros((1, _D), jnp.float32)

    def _row_copy(b):
        return pltpu.make_async_copy(
            zrow, out_hbm.at[b, pl.ds(idx_ref[b], 1)], row_sem
        )

    def _pass(start):
        def body(b, _):
            dropped = drop_ref[b] > (1.0 - 0.2)

            @pl.when(dropped)
            def _():
                cp = _row_copy(b)
                if start:
                    cp.start()
                else:
                    cp.wait()

            return 0

        jax.lax.fori_loop(0, _BS, body, 0)

    _pass(start=True)
    _pass(start=False)


@jax.jit
def kernel(inputs, index, drop_rand):
    return pl.pallas_call(
        _body,
        grid=(),
        in_specs=[
            pl.BlockSpec(memory_space=pltpu.SMEM),
            pl.BlockSpec(memory_space=pltpu.SMEM),
            pl.BlockSpec(memory_space=pl.ANY),
        ],
        out_specs=pl.BlockSpec(memory_space=pl.ANY),
        out_shape=jax.ShapeDtypeStruct((_BS, _L, _D), jnp.float32),
        scratch_shapes=[
            pltpu.VMEM((1, _D), jnp.float32),
            pltpu.SemaphoreType.DMA,
        ],
        input_output_aliases={2: 0},
    )(index, drop_rand, inputs)


# speedup vs baseline: 3.7289x; 2.1936x over previous
import jax
import jax.numpy as jnp
from jax.experimental import pallas as pl
from jax.experimental.pallas import tpu as pltpu

_BS, _L, _D = 128, 12, 8192


def _body(idx_ref, drop_ref, in_ref, out_ref):
    l = pl.program_id(0)
    out_ref[...] = in_ref[...]

    def patch(b, _):
        dropped = jnp.logical_and(
            drop_ref[b] > (1.0 - 0.2), idx_ref[b] == l
        )

        @pl.when(dropped)
        def _():
            out_ref[0, pl.ds(b, 1), :] = jnp.zeros((1, _D), jnp.float32)

        return 0

    jax.lax.fori_loop(0, _BS, patch, 0)


def _transposed_call(index, drop_rand, x_t):
    grid_spec = pltpu.PrefetchScalarGridSpec(
        num_scalar_prefetch=2,
        grid=(_L,),
        in_specs=[
            pl.BlockSpec((1, _BS, _D), lambda l, idx_ref, drop_ref: (l, 0, 0)),
        ],
        out_specs=pl.BlockSpec((1, _BS, _D), lambda l, idx_ref, drop_ref: (l, 0, 0)),
    )
    return pl.pallas_call(
        _body,
        grid_spec=grid_spec,
        out_shape=jax.ShapeDtypeStruct((_L, _BS, _D), jnp.float32),
        compiler_params=pltpu.CompilerParams(
            dimension_semantics=("arbitrary",),
        ),
    )(index, drop_rand, x_t)


@jax.jit
def kernel(inputs, index, drop_rand):
    x_t = jnp.transpose(inputs, (1, 0, 2))
    out_t = _transposed_call(index, drop_rand, x_t)
    return jnp.transpose(out_t, (1, 0, 2))


# speedup vs baseline: 4.0339x; 1.0818x over previous
import jax
import jax.numpy as jnp
from jax.experimental import pallas as pl
from jax.experimental.pallas import tpu as pltpu

_BS, _L, _D = 128, 12, 8192


def _body(idx_ref, drop_ref, in_ref, out_ref):
    l0 = pl.program_id(0) * 2
    out_ref[...] = in_ref[...]

    def patch(b, _):
        dropped = drop_ref[b] > (1.0 - 0.2)
        for k in range(2):

            @pl.when(jnp.logical_and(dropped, idx_ref[b] == l0 + k))
            def _():
                out_ref[k, pl.ds(b, 1), :] = jnp.zeros((1, _D), jnp.float32)

        return 0

    jax.lax.fori_loop(0, _BS, patch, 0)


def _transposed_call(index, drop_rand, x_t):
    grid_spec = pltpu.PrefetchScalarGridSpec(
        num_scalar_prefetch=2,
        grid=(_L // 2,),
        in_specs=[
            pl.BlockSpec((2, _BS, _D), lambda l, idx_ref, drop_ref: (l, 0, 0)),
        ],
        out_specs=pl.BlockSpec((2, _BS, _D), lambda l, idx_ref, drop_ref: (l, 0, 0)),
    )
    return pl.pallas_call(
        _body,
        grid_spec=grid_spec,
        out_shape=jax.ShapeDtypeStruct((_L, _BS, _D), jnp.float32),
        compiler_params=pltpu.CompilerParams(
            dimension_semantics=("arbitrary",),
        ),
    )(index, drop_rand, x_t)


@jax.jit
def kernel(inputs, index, drop_rand):
    x_t = jnp.transpose(inputs, (1, 0, 2))
    out_t = _transposed_call(index, drop_rand, x_t)
    return jnp.transpose(out_t, (1, 0, 2))
